# baseline (speedup 1.0000x reference)
import functools

import jax
import jax.numpy as jnp
from jax import lax
from jax.experimental import pallas as pl
from jax.experimental.pallas import tpu as pltpu
from jax.experimental.pallas import tpu_sc as plsc


def _max_arg0(x):
    r, c = x.shape
    grp = 16
    g = r // grp
    x3 = x.reshape(grp, g, c)
    v = x3[0]
    ix = jnp.zeros((g, c), jnp.int32)
    for j in range(1, grp):
        xj = x3[j]
        better = xj > v
        v = jnp.where(better, xj, v)
        ix = jnp.where(better, j, ix)
    vmax = jnp.max(v, axis=0)
    iota_r = lax.broadcasted_iota(jnp.int32, (g, c), 0)
    cand = jnp.where(v == vmax[None, :], ix * g + iota_r, r)
    arg = jnp.min(cand, axis=0)
    return vmax, arg


def _sim_argmax_body(nblk, ni, d0_ref, d1_ref, m0_ref, s0_ref, m1_ref, s1_ref):
    i = pl.program_id(1)
    a = d0_ref[0]
    b = d1_ref[0]
    s = lax.dot_general(
        a, b, (((0,), (0,)), ((), ())),
        preferred_element_type=jnp.float32,
    )
    st = lax.dot_general(
        b, a, (((0,), (0,)), ((), ())),
        preferred_element_type=jnp.float32,
    )
    m = s.shape[1]

    rowmax, rowarg = _max_arg0(st)
    m0_ref[0, 0] = rowarg
    s0_ref[0, 0] = (rowmax + 1.0) * 0.5

    colmax, colarg = _max_arg0(s)
    colarg = colarg + i * nblk

    @pl.when(i == 0)
    def _():
        m1_ref[0, 0] = colarg
        s1_ref[0, 0] = colmax

    @pl.when(i > 0)
    def _():
        prev = s1_ref[0, 0]
        better = colmax > prev
        m1_ref[0, 0] = jnp.where(better, colarg, m1_ref[0, 0])
        s1_ref[0, 0] = jnp.where(better, colmax, prev)

    @pl.when(i == ni - 1)
    def _():
        s1_ref[0, 0] = (s1_ref[0, 0] + 1.0) * 0.5


def _sim_argmax(d0, d1, nblk=2048):
    bsz, d, n = d0.shape
    m = d1.shape[2]
    ni = n // nblk
    out3 = lambda dt: jax.ShapeDtypeStruct((bsz, 1, m), dt)
    grid = (bsz, ni)
    return pl.pallas_call(
        functools.partial(_sim_argmax_body, nblk, ni),
        grid=grid,
        in_specs=[
            pl.BlockSpec((1, d, nblk), lambda b, i: (b, 0, i)),
            pl.BlockSpec((1, d, m), lambda b, i: (b, 0, 0)),
        ],
        out_specs=[
            pl.BlockSpec((1, 1, nblk), lambda b, i: (b, 0, i)),
            pl.BlockSpec((1, 1, nblk), lambda b, i: (b, 0, i)),
            pl.BlockSpec((1, 1, m), lambda b, i: (b, 0, 0)),
            pl.BlockSpec((1, 1, m), lambda b, i: (b, 0, 0)),
        ],
        out_shape=[
            jax.ShapeDtypeStruct((bsz, 1, n), jnp.int32),
            jax.ShapeDtypeStruct((bsz, 1, n), jnp.float32),
            out3(jnp.int32),
            out3(jnp.float32),
        ],
        compiler_params=pltpu.CompilerParams(
            dimension_semantics=("parallel", "arbitrary"),
        ),
    )(d0, d1)


def _mutual_sc(m0_flat, m1_flat, per_b):
    info = plsc.get_sparse_core_info()
    nc, ns, lanes = info.num_cores, info.num_subcores, info.num_lanes
    nw = nc * ns
    tot = m0_flat.shape[0]
    ch = tot // nw

    mesh = plsc.VectorSubcoreMesh(core_axis_name="c", subcore_axis_name="s")

    @functools.partial(
        pl.kernel, mesh=mesh,
        compiler_params=pltpu.CompilerParams(needs_layout_passes=False),
        out_type=(
            jax.ShapeDtypeStruct((tot,), jnp.int32),
            jax.ShapeDtypeStruct((tot,), jnp.int32),
        ),
        scratch_types=[
            pltpu.VMEM((per_b,), jnp.int32),
            pltpu.VMEM((per_b,), jnp.int32),
            pltpu.VMEM((ch,), jnp.int32),
            pltpu.VMEM((ch,), jnp.int32),
            pltpu.SemaphoreType.DMA,
        ],
    )
    def k(m0_hbm, m1_hbm, o0_hbm, o1_hbm, t0, t1, o0, o1, sem):
        wid = lax.axis_index("s") * nc + lax.axis_index("c")
        base = wid * ch
        boff = (base // per_b) * per_b
        cps = [
            pltpu.make_async_copy(m0_hbm.at[pl.ds(boff, per_b)], t0, sem),
            pltpu.make_async_copy(m1_hbm.at[pl.ds(boff, per_b)], t1, sem),
        ]
        for cp in cps:
            cp.start()
        for cp in cps:
            cp.wait()
        local = base - boff
        neg1 = jnp.full((lanes,), -1, jnp.int32)
        for j in range(ch // lanes):
            off = j * lanes
            pos = lax.iota(jnp.int32, lanes) + off
            pos = pos + local
            idx0 = t0[pl.ds(local + off, lanes)]
            back0 = plsc.load_gather(t1, [idx0])
            o0[pl.ds(off, lanes)] = jnp.where(back0 == pos, idx0, neg1)
            idx1 = t1[pl.ds(local + off, lanes)]
            back1 = plsc.load_gather(t0, [idx1])
            o1[pl.ds(off, lanes)] = jnp.where(back1 == pos, idx1, neg1)
        pltpu.sync_copy(o0, o0_hbm.at[pl.ds(base, ch)])
        pltpu.sync_copy(o1, o1_hbm.at[pl.ds(base, ch)])

    return k(m0_flat, m1_flat)


def kernel(descriptors0, descriptors1):
    bsz, d, n = descriptors0.shape
    m = descriptors1.shape[2]
    m0r, s0, m1r, s1 = _sim_argmax(descriptors0, descriptors1)
    m0f, m1f = _mutual_sc(m0r.reshape(bsz * n), m1r.reshape(bsz * m), n)
    return (
        m0f.reshape(bsz, n),
        m1f.reshape(bsz, m),
        s0.reshape(bsz, n),
        s1.reshape(bsz, m),
    )


# speedup vs baseline: 1.0750x; 1.0750x over previous
import functools

import jax
import jax.numpy as jnp
from jax import lax
from jax.experimental import pallas as pl
from jax.experimental.pallas import tpu as pltpu
from jax.experimental.pallas import tpu_sc as plsc


def _max_arg0(x):
    r, c = x.shape
    grp = 256
    g = r // grp
    x3 = x.reshape(grp, g, c)
    v = x3[0]
    ix = jnp.zeros((g, c), jnp.int32)
    for j in range(1, grp):
        xj = x3[j]
        better = xj > v
        v = jnp.where(better, xj, v)
        ix = jnp.where(better, j, ix)
    vmax = jnp.max(v, axis=0)
    iota_r = lax.broadcasted_iota(jnp.int32, (g, c), 0)
    cand = jnp.where(v == vmax[None, :], ix * g + iota_r, r)
    arg = jnp.min(cand, axis=0)
    return vmax, arg


def _sim_argmax_body(nblk, ni, d0_ref, d1_ref, m0_ref, s0_ref, m1_ref, s1_ref):
    i = pl.program_id(1)
    a = d0_ref[0]
    b = d1_ref[0]
    s = lax.dot_general(
        a, b, (((0,), (0,)), ((), ())),
        preferred_element_type=jnp.float32,
    )
    st = lax.dot_general(
        b, a, (((0,), (0,)), ((), ())),
        preferred_element_type=jnp.float32,
    )
    m = s.shape[1]

    rowmax, rowarg = _max_arg0(st)
    m0_ref[0, 0] = rowarg
    s0_ref[0, 0] = (rowmax + 1.0) * 0.5

    colmax, colarg = _max_arg0(s)
    colarg = colarg + i * nblk

    @pl.when(i == 0)
    def _():
        m1_ref[0, 0] = colarg
        s1_ref[0, 0] = colmax

    @pl.when(i > 0)
    def _():
        prev = s1_ref[0, 0]
        better = colmax > prev
        m1_ref[0, 0] = jnp.where(better, colarg, m1_ref[0, 0])
        s1_ref[0, 0] = jnp.where(better, colmax, prev)

    @pl.when(i == ni - 1)
    def _():
        s1_ref[0, 0] = (s1_ref[0, 0] + 1.0) * 0.5


def _sim_argmax(d0, d1, nblk=2048):
    bsz, d, n = d0.shape
    m = d1.shape[2]
    ni = n // nblk
    out3 = lambda dt: jax.ShapeDtypeStruct((bsz, 1, m), dt)
    grid = (bsz, ni)
    return pl.pallas_call(
        functools.partial(_sim_argmax_body, nblk, ni),
        grid=grid,
        in_specs=[
            pl.BlockSpec((1, d, nblk), lambda b, i: (b, 0, i)),
            pl.BlockSpec((1, d, m), lambda b, i: (b, 0, 0)),
        ],
        out_specs=[
            pl.BlockSpec((1, 1, nblk), lambda b, i: (b, 0, i)),
            pl.BlockSpec((1, 1, nblk), lambda b, i: (b, 0, i)),
            pl.BlockSpec((1, 1, m), lambda b, i: (b, 0, 0)),
            pl.BlockSpec((1, 1, m), lambda b, i: (b, 0, 0)),
        ],
        out_shape=[
            jax.ShapeDtypeStruct((bsz, 1, n), jnp.int32),
            jax.ShapeDtypeStruct((bsz, 1, n), jnp.float32),
            out3(jnp.int32),
            out3(jnp.float32),
        ],
        compiler_params=pltpu.CompilerParams(
            dimension_semantics=("parallel", "arbitrary"),
        ),
    )(d0, d1)


def _mutual_sc(m0_flat, m1_flat, per_b):
    info = plsc.get_sparse_core_info()
    nc, ns, lanes = info.num_cores, info.num_subcores, info.num_lanes
    nw = nc * ns
    tot = m0_flat.shape[0]
    ch = tot // nw

    mesh = plsc.VectorSubcoreMesh(core_axis_name="c", subcore_axis_name="s")

    @functools.partial(
        pl.kernel, mesh=mesh,
        compiler_params=pltpu.CompilerParams(needs_layout_passes=False),
        out_type=(
            jax.ShapeDtypeStruct((tot,), jnp.int32),
            jax.ShapeDtypeStruct((tot,), jnp.int32),
        ),
        scratch_types=[
            pltpu.VMEM((per_b,), jnp.int32),
            pltpu.VMEM((per_b,), jnp.int32),
            pltpu.VMEM((ch,), jnp.int32),
            pltpu.VMEM((ch,), jnp.int32),
            pltpu.SemaphoreType.DMA,
        ],
    )
    def k(m0_hbm, m1_hbm, o0_hbm, o1_hbm, t0, t1, o0, o1, sem):
        wid = lax.axis_index("s") * nc + lax.axis_index("c")
        base = wid * ch
        boff = (base // per_b) * per_b
        cps = [
            pltpu.make_async_copy(m0_hbm.at[pl.ds(boff, per_b)], t0, sem),
            pltpu.make_async_copy(m1_hbm.at[pl.ds(boff, per_b)], t1, sem),
        ]
        for cp in cps:
            cp.start()
        for cp in cps:
            cp.wait()
        local = base - boff
        neg1 = jnp.full((lanes,), -1, jnp.int32)
        for j in range(ch // lanes):
            off = j * lanes
            pos = lax.iota(jnp.int32, lanes) + off
            pos = pos + local
            idx0 = t0[pl.ds(local + off, lanes)]
            back0 = plsc.load_gather(t1, [idx0])
            o0[pl.ds(off, lanes)] = jnp.where(back0 == pos, idx0, neg1)
            idx1 = t1[pl.ds(local + off, lanes)]
            back1 = plsc.load_gather(t0, [idx1])
            o1[pl.ds(off, lanes)] = jnp.where(back1 == pos, idx1, neg1)
        pltpu.sync_copy(o0, o0_hbm.at[pl.ds(base, ch)])
        pltpu.sync_copy(o1, o1_hbm.at[pl.ds(base, ch)])

    return k(m0_flat, m1_flat)


def kernel(descriptors0, descriptors1):
    bsz, d, n = descriptors0.shape
    m = descriptors1.shape[2]
    m0r, s0, m1r, s1 = _sim_argmax(descriptors0, descriptors1)
    m0f, m1f = _mutual_sc(m0r.reshape(bsz * n), m1r.reshape(bsz * m), n)
    return (
        m0f.reshape(bsz, n),
        m1f.reshape(bsz, m),
        s0.reshape(bsz, n),
        s1.reshape(bsz, m),
    )
